# baseline (speedup 1.0000x reference)
import math

import jax
import jax.numpy as jnp
from jax.experimental import pallas as pl
from jax.experimental.pallas import tpu as pltpu

_BM = 512


def _scores_body(temp_ref, srcT_ref, tgt_emb_ref, tgt_ext_ref,
                 corres_ref, weight_ref, yg_ref):
    n_cols = tgt_emb_ref.shape[2]
    dot = jax.lax.dot_general(
        srcT_ref[0].astype(jnp.bfloat16), tgt_emb_ref[0].astype(jnp.bfloat16),
        (((1,), (0,)), ((), ())),
        preferred_element_type=jnp.float32)
    inv_sqrt_d = jnp.float32(1.0 / math.sqrt(srcT_ref.shape[2]))
    z = (dot * inv_sqrt_d) * temp_ref[pl.program_id(0), 0]
    zmax = jnp.max(z, axis=1, keepdims=True)
    ssum = jnp.sum(jnp.exp(z - zmax), axis=1, keepdims=True)
    w = 1.0 / ssum
    col = jax.lax.broadcasted_iota(jnp.int32, z.shape, 1)
    idx = jnp.min(jnp.where(z == zmax, col, n_cols), axis=1, keepdims=True)
    corres_ref[0] = idx
    weight_ref[0] = w
    oh = jnp.where(col == idx, jnp.float32(1.0), jnp.float32(0.0))
    yg_ref[0] = jax.lax.dot_general(
        tgt_ext_ref[0], oh, (((1,), (1,)), ((), ())),
        preferred_element_type=jnp.float32,
        precision=jax.lax.Precision.HIGHEST)


def _moments_body(yg_ref, xext_ref, w_ref, cov_ref, mu_ref):
    eps = jnp.float32(1e-7)
    y = yg_ref[0]
    x = xext_ref[0]
    w = w_ref[0]
    tw = jnp.sum(w)
    nw = w / (tw + eps)
    mu_y = jnp.sum(nw * y, axis=1, keepdims=True)
    mu_x = jnp.sum(nw * x, axis=1, keepdims=True)
    yc = y - mu_y
    xw = nw * (x - mu_x)
    cov_ref[0] = jax.lax.dot_general(
        yc, xw, (((1,), (1,)), ((), ())),
        preferred_element_type=jnp.float32,
        precision=jax.lax.Precision.HIGHEST)
    mu_ref[0] = jnp.concatenate([mu_y, mu_x] + [jnp.zeros_like(mu_y)] * 6,
                                axis=1)


def _best_rotation(cov):
    mxx, mxy, mxz = cov[:, 0, 0], cov[:, 1, 0], cov[:, 2, 0]
    myx, myy, myz = cov[:, 0, 1], cov[:, 1, 1], cov[:, 2, 1]
    mzx, mzy, mzz = cov[:, 0, 2], cov[:, 1, 2], cov[:, 2, 2]
    a = {
        (0, 0): mxx + myy + mzz, (0, 1): myz - mzy,
        (0, 2): mzx - mxz, (0, 3): mxy - myx,
        (1, 1): mxx - myy - mzz, (1, 2): mxy + myx, (1, 3): mzx + mxz,
        (2, 2): myy - mxx - mzz, (2, 3): myz + mzy,
        (3, 3): mzz - mxx - myy,
    }
    one = jnp.ones_like(mxx)
    zero = jnp.zeros_like(mxx)
    v = [[one if i == j else zero for j in range(4)] for i in range(4)]

    def ga(i, j):
        return a[(i, j)] if i <= j else a[(j, i)]

    def sa(i, j, val):
        a[(i, j) if i <= j else (j, i)] = val

    tiny = jnp.float32(1e-30)
    for _ in range(6):
        for (p, q) in ((0, 1), (0, 2), (0, 3), (1, 2), (1, 3), (2, 3)):
            app, aqq, apq = ga(p, p), ga(q, q), ga(p, q)
            guard = jnp.abs(apq) < tiny
            tau = (aqq - app) / (2.0 * jnp.where(guard, one, apq))
            t = jnp.sign(tau) / (jnp.abs(tau) + jnp.sqrt(1.0 + tau * tau))
            t = jnp.where(guard, zero, t)
            c = 1.0 / jnp.sqrt(1.0 + t * t)
            s = t * c
            sa(p, p, app - t * apq)
            sa(q, q, aqq + t * apq)
            sa(p, q, zero)
            for k in range(4):
                if k == p or k == q:
                    continue
                akp, akq = ga(k, p), ga(k, q)
                sa(k, p, c * akp - s * akq)
                sa(k, q, s * akp + c * akq)
            for k in range(4):
                vkp, vkq = v[k][p], v[k][q]
                v[k][p] = c * vkp - s * vkq
                v[k][q] = s * vkp + c * vkq
    best_lam = ga(0, 0)
    qv = [v[k][0] for k in range(4)]
    for i in range(1, 4):
        lam_i = ga(i, i)
        cond = lam_i > best_lam
        qv = [jnp.where(cond, v[k][i], qv[k]) for k in range(4)]
        best_lam = jnp.where(cond, lam_i, best_lam)
    inv_n = 1.0 / jnp.sqrt(qv[0] ** 2 + qv[1] ** 2 + qv[2] ** 2 + qv[3] ** 2)
    qw, qx, qy, qz = (c_ * inv_n for c_ in qv)
    R = jnp.stack([
        jnp.stack([1 - 2 * (qy * qy + qz * qz), 2 * (qx * qy - qw * qz),
                   2 * (qx * qz + qw * qy)], axis=-1),
        jnp.stack([2 * (qx * qy + qw * qz), 1 - 2 * (qx * qx + qz * qz),
                   2 * (qy * qz - qw * qx)], axis=-1),
        jnp.stack([2 * (qx * qz - qw * qy), 2 * (qy * qz + qw * qx),
                   1 - 2 * (qx * qx + qy * qy)], axis=-1),
    ], axis=-2)
    return R


def kernel(src_embedding, tgt_embedding, src, tgt, temperature, is_corr):
    B, D, N = src_embedding.shape
    bm = _BM
    nb = N // bm
    srcT = jnp.transpose(src_embedding, (0, 2, 1))
    ones = jnp.ones((B, 1, N), jnp.float32)
    zeros = jnp.zeros((B, 4, N), jnp.float32)
    src_ext = jnp.concatenate([src, ones, zeros], axis=1)
    tgt_ext = jnp.concatenate([tgt, ones, zeros], axis=1)

    corres2, weight2, yg = pl.pallas_call(
        _scores_body,
        grid=(B, nb),
        in_specs=[
            pl.BlockSpec((B, 1), lambda b, n: (0, 0),
                         memory_space=pltpu.SMEM),
            pl.BlockSpec((1, bm, D), lambda b, n: (b, n, 0)),
            pl.BlockSpec((1, D, N), lambda b, n: (b, 0, 0)),
            pl.BlockSpec((1, 8, N), lambda b, n: (b, 0, 0)),
        ],
        out_specs=[
            pl.BlockSpec((1, bm, 1), lambda b, n: (b, n, 0)),
            pl.BlockSpec((1, bm, 1), lambda b, n: (b, n, 0)),
            pl.BlockSpec((1, 8, bm), lambda b, n: (b, 0, n)),
        ],
        out_shape=[
            jax.ShapeDtypeStruct((B, N, 1), jnp.int32),
            jax.ShapeDtypeStruct((B, N, 1), jnp.float32),
            jax.ShapeDtypeStruct((B, 8, N), jnp.float32),
        ],
        compiler_params=pltpu.CompilerParams(
            dimension_semantics=("parallel", "arbitrary")),
    )(temperature.astype(jnp.float32).reshape(B, 1), srcT, tgt_embedding,
      tgt_ext)

    cov8, mu8 = pl.pallas_call(
        _moments_body,
        grid=(B,),
        in_specs=[
            pl.BlockSpec((1, 8, N), lambda b: (b, 0, 0)),
            pl.BlockSpec((1, 8, N), lambda b: (b, 0, 0)),
            pl.BlockSpec((1, 1, N), lambda b: (b, 0, 0)),
        ],
        out_specs=[
            pl.BlockSpec((1, 8, 8), lambda b: (b, 0, 0)),
            pl.BlockSpec((1, 8, 8), lambda b: (b, 0, 0)),
        ],
        out_shape=[
            jax.ShapeDtypeStruct((B, 8, 8), jnp.float32),
            jax.ShapeDtypeStruct((B, 8, 8), jnp.float32),
        ],
        compiler_params=pltpu.CompilerParams(
            dimension_semantics=("arbitrary",)),
    )(yg, src_ext, weight2.reshape(B, 1, N))

    cov = cov8[:, :3, :3]
    mu_y = mu8[:, :3, 0]
    mu_x = mu8[:, :3, 1]
    R = _best_rotation(cov)
    rmux = jnp.sum(R * mu_x[:, None, :], axis=2)
    T = (mu_y[:, None, :] - rmux[:, :, None]).astype(jnp.float32)
    return (R.astype(jnp.float32), T, corres2, weight2)


# speedup vs baseline: 1.7210x; 1.7210x over previous
import math

import jax
import jax.numpy as jnp
from jax.experimental import pallas as pl
from jax.experimental.pallas import tpu as pltpu

_BM = 512


def _scores_body(temp_ref, srcT_ref, tgt_emb_ref, tgt_ext_ref,
                 corres_ref, weight_ref, yg_ref):
    n_cols = tgt_emb_ref.shape[2]
    dot = jax.lax.dot_general(
        srcT_ref[0].astype(jnp.bfloat16), tgt_emb_ref[0].astype(jnp.bfloat16),
        (((1,), (0,)), ((), ())),
        preferred_element_type=jnp.float32)
    inv_sqrt_d = jnp.float32(1.0 / math.sqrt(srcT_ref.shape[2]))
    z = (dot * inv_sqrt_d) * temp_ref[pl.program_id(0), 0]
    zmax = jnp.max(z, axis=1, keepdims=True)
    ssum = jnp.sum(jnp.exp(z - zmax), axis=1, keepdims=True)
    w = 1.0 / ssum
    col = jax.lax.broadcasted_iota(jnp.int32, z.shape, 1)
    idx = jnp.min(jnp.where(z == zmax, col, n_cols), axis=1, keepdims=True)
    corres_ref[0] = idx
    weight_ref[0] = w
    oh = jnp.where(col == idx, jnp.float32(1.0),
                   jnp.float32(0.0)).astype(jnp.bfloat16)
    yg16 = jax.lax.dot_general(
        tgt_ext_ref[0], oh, (((1,), (1,)), ((), ())),
        preferred_element_type=jnp.float32)
    yg_ref[0] = yg16[:8] + yg16[8:]


def _moments_body(yg_ref, xext_ref, w_ref, cov_ref, mu_ref):
    eps = jnp.float32(1e-7)
    y = yg_ref[0]
    x = xext_ref[0]
    w = w_ref[0]
    tw = jnp.sum(w)
    nw = w / (tw + eps)
    mu_y = jnp.sum(nw * y, axis=1, keepdims=True)
    mu_x = jnp.sum(nw * x, axis=1, keepdims=True)
    yc = y - mu_y
    xw = nw * (x - mu_x)
    cov_ref[0] = jax.lax.dot_general(
        yc, xw, (((1,), (1,)), ((), ())),
        preferred_element_type=jnp.float32,
        precision=jax.lax.Precision.HIGHEST)
    mu_ref[0] = jnp.concatenate([mu_y, mu_x] + [jnp.zeros_like(mu_y)] * 6,
                                axis=1)


def _best_rotation(cov):
    mxx, mxy, mxz = cov[:, 0, 0], cov[:, 1, 0], cov[:, 2, 0]
    myx, myy, myz = cov[:, 0, 1], cov[:, 1, 1], cov[:, 2, 1]
    mzx, mzy, mzz = cov[:, 0, 2], cov[:, 1, 2], cov[:, 2, 2]
    a = {
        (0, 0): mxx + myy + mzz, (0, 1): myz - mzy,
        (0, 2): mzx - mxz, (0, 3): mxy - myx,
        (1, 1): mxx - myy - mzz, (1, 2): mxy + myx, (1, 3): mzx + mxz,
        (2, 2): myy - mxx - mzz, (2, 3): myz + mzy,
        (3, 3): mzz - mxx - myy,
    }
    one = jnp.ones_like(mxx)
    zero = jnp.zeros_like(mxx)
    v = [[one if i == j else zero for j in range(4)] for i in range(4)]

    def ga(i, j):
        return a[(i, j)] if i <= j else a[(j, i)]

    def sa(i, j, val):
        a[(i, j) if i <= j else (j, i)] = val

    tiny = jnp.float32(1e-30)
    for _ in range(6):
        for (p, q) in ((0, 1), (0, 2), (0, 3), (1, 2), (1, 3), (2, 3)):
            app, aqq, apq = ga(p, p), ga(q, q), ga(p, q)
            guard = jnp.abs(apq) < tiny
            tau = (aqq - app) / (2.0 * jnp.where(guard, one, apq))
            t = jnp.sign(tau) / (jnp.abs(tau) + jnp.sqrt(1.0 + tau * tau))
            t = jnp.where(guard, zero, t)
            c = 1.0 / jnp.sqrt(1.0 + t * t)
            s = t * c
            sa(p, p, app - t * apq)
            sa(q, q, aqq + t * apq)
            sa(p, q, zero)
            for k in range(4):
                if k == p or k == q:
                    continue
                akp, akq = ga(k, p), ga(k, q)
                sa(k, p, c * akp - s * akq)
                sa(k, q, s * akp + c * akq)
            for k in range(4):
                vkp, vkq = v[k][p], v[k][q]
                v[k][p] = c * vkp - s * vkq
                v[k][q] = s * vkp + c * vkq
    best_lam = ga(0, 0)
    qv = [v[k][0] for k in range(4)]
    for i in range(1, 4):
        lam_i = ga(i, i)
        cond = lam_i > best_lam
        qv = [jnp.where(cond, v[k][i], qv[k]) for k in range(4)]
        best_lam = jnp.where(cond, lam_i, best_lam)
    inv_n = 1.0 / jnp.sqrt(qv[0] ** 2 + qv[1] ** 2 + qv[2] ** 2 + qv[3] ** 2)
    qw, qx, qy, qz = (c_ * inv_n for c_ in qv)
    R = jnp.stack([
        jnp.stack([1 - 2 * (qy * qy + qz * qz), 2 * (qx * qy - qw * qz),
                   2 * (qx * qz + qw * qy)], axis=-1),
        jnp.stack([2 * (qx * qy + qw * qz), 1 - 2 * (qx * qx + qz * qz),
                   2 * (qy * qz - qw * qx)], axis=-1),
        jnp.stack([2 * (qx * qz - qw * qy), 2 * (qy * qz + qw * qx),
                   1 - 2 * (qx * qx + qy * qy)], axis=-1),
    ], axis=-2)
    return R


def kernel(src_embedding, tgt_embedding, src, tgt, temperature, is_corr):
    B, D, N = src_embedding.shape
    bm = _BM
    nb = N // bm
    srcT = jnp.transpose(src_embedding, (0, 2, 1))
    ones = jnp.ones((B, 1, N), jnp.float32)
    zeros = jnp.zeros((B, 4, N), jnp.float32)
    src_ext = jnp.concatenate([src, ones, zeros], axis=1)
    tgt_ext = jnp.concatenate([tgt, ones, zeros], axis=1)
    tgt_hi = tgt_ext.astype(jnp.bfloat16)
    tgt_lo = (tgt_ext - tgt_hi.astype(jnp.float32)).astype(jnp.bfloat16)
    tgt16 = jnp.concatenate([tgt_hi, tgt_lo], axis=1)

    corres2, weight2, yg = pl.pallas_call(
        _scores_body,
        grid=(B, nb),
        in_specs=[
            pl.BlockSpec((B, 1), lambda b, n: (0, 0),
                         memory_space=pltpu.SMEM),
            pl.BlockSpec((1, bm, D), lambda b, n: (b, n, 0)),
            pl.BlockSpec((1, D, N), lambda b, n: (b, 0, 0)),
            pl.BlockSpec((1, 16, N), lambda b, n: (b, 0, 0)),
        ],
        out_specs=[
            pl.BlockSpec((1, bm, 1), lambda b, n: (b, n, 0)),
            pl.BlockSpec((1, bm, 1), lambda b, n: (b, n, 0)),
            pl.BlockSpec((1, 8, bm), lambda b, n: (b, 0, n)),
        ],
        out_shape=[
            jax.ShapeDtypeStruct((B, N, 1), jnp.int32),
            jax.ShapeDtypeStruct((B, N, 1), jnp.float32),
            jax.ShapeDtypeStruct((B, 8, N), jnp.float32),
        ],
        compiler_params=pltpu.CompilerParams(
            dimension_semantics=("parallel", "arbitrary")),
    )(temperature.astype(jnp.float32).reshape(B, 1), srcT, tgt_embedding,
      tgt16)

    cov8, mu8 = pl.pallas_call(
        _moments_body,
        grid=(B,),
        in_specs=[
            pl.BlockSpec((1, 8, N), lambda b: (b, 0, 0)),
            pl.BlockSpec((1, 8, N), lambda b: (b, 0, 0)),
            pl.BlockSpec((1, 1, N), lambda b: (b, 0, 0)),
        ],
        out_specs=[
            pl.BlockSpec((1, 8, 8), lambda b: (b, 0, 0)),
            pl.BlockSpec((1, 8, 8), lambda b: (b, 0, 0)),
        ],
        out_shape=[
            jax.ShapeDtypeStruct((B, 8, 8), jnp.float32),
            jax.ShapeDtypeStruct((B, 8, 8), jnp.float32),
        ],
        compiler_params=pltpu.CompilerParams(
            dimension_semantics=("arbitrary",)),
    )(yg, src_ext, weight2.reshape(B, 1, N))

    cov = cov8[:, :3, :3]
    mu_y = mu8[:, :3, 0]
    mu_x = mu8[:, :3, 1]
    R = _best_rotation(cov)
    rmux = jnp.sum(R * mu_x[:, None, :], axis=2)
    T = (mu_y[:, None, :] - rmux[:, :, None]).astype(jnp.float32)
    return (R.astype(jnp.float32), T, corres2, weight2)


# speedup vs baseline: 1.7219x; 1.0005x over previous
import math

import jax
import jax.numpy as jnp
from jax.experimental import pallas as pl
from jax.experimental.pallas import tpu as pltpu

_BM = 512


def _scores_body(temp_ref, srcT_ref, tgt_emb_ref, tgt_ext_ref,
                 corres_ref, weight_ref, yg_ref):
    n_cols = tgt_emb_ref.shape[2]
    dot = jax.lax.dot_general(
        srcT_ref[0], tgt_emb_ref[0],
        (((0,), (0,)), ((), ())),
        preferred_element_type=jnp.float32)
    inv_sqrt_d = jnp.float32(1.0 / math.sqrt(srcT_ref.shape[1]))
    z = (dot * inv_sqrt_d) * temp_ref[pl.program_id(0), 0]
    zmax = jnp.max(z, axis=1, keepdims=True)
    ssum = jnp.sum(jnp.exp(z - zmax), axis=1, keepdims=True)
    w = 1.0 / ssum
    col = jax.lax.broadcasted_iota(jnp.int32, z.shape, 1)
    idx = jnp.min(jnp.where(z == zmax, col, n_cols), axis=1, keepdims=True)
    corres_ref[0] = idx
    weight_ref[0] = w
    oh = jnp.where(col == idx, jnp.float32(1.0),
                   jnp.float32(0.0)).astype(jnp.bfloat16)
    yg16 = jax.lax.dot_general(
        tgt_ext_ref[0], oh, (((1,), (1,)), ((), ())),
        preferred_element_type=jnp.float32)
    yg_ref[0] = yg16[:8] + yg16[8:]


def _moments_body(yg_ref, xext_ref, w_ref, cov_ref, mu_ref):
    eps = jnp.float32(1e-7)
    y = yg_ref[0]
    x = xext_ref[0]
    w = w_ref[0]
    tw = jnp.sum(w)
    nw = w / (tw + eps)
    mu_y = jnp.sum(nw * y, axis=1, keepdims=True)
    mu_x = jnp.sum(nw * x, axis=1, keepdims=True)
    yc = y - mu_y
    xw = nw * (x - mu_x)
    cov_ref[0] = jax.lax.dot_general(
        yc, xw, (((1,), (1,)), ((), ())),
        preferred_element_type=jnp.float32,
        precision=jax.lax.Precision.HIGHEST)
    mu_ref[0] = jnp.concatenate([mu_y, mu_x] + [jnp.zeros_like(mu_y)] * 6,
                                axis=1)


def _best_rotation(cov):
    mxx, mxy, mxz = cov[:, 0, 0], cov[:, 1, 0], cov[:, 2, 0]
    myx, myy, myz = cov[:, 0, 1], cov[:, 1, 1], cov[:, 2, 1]
    mzx, mzy, mzz = cov[:, 0, 2], cov[:, 1, 2], cov[:, 2, 2]
    a = {
        (0, 0): mxx + myy + mzz, (0, 1): myz - mzy,
        (0, 2): mzx - mxz, (0, 3): mxy - myx,
        (1, 1): mxx - myy - mzz, (1, 2): mxy + myx, (1, 3): mzx + mxz,
        (2, 2): myy - mxx - mzz, (2, 3): myz + mzy,
        (3, 3): mzz - mxx - myy,
    }
    one = jnp.ones_like(mxx)
    zero = jnp.zeros_like(mxx)
    v = [[one if i == j else zero for j in range(4)] for i in range(4)]

    def ga(i, j):
        return a[(i, j)] if i <= j else a[(j, i)]

    def sa(i, j, val):
        a[(i, j) if i <= j else (j, i)] = val

    tiny = jnp.float32(1e-30)
    for _ in range(6):
        for (p, q) in ((0, 1), (0, 2), (0, 3), (1, 2), (1, 3), (2, 3)):
            app, aqq, apq = ga(p, p), ga(q, q), ga(p, q)
            guard = jnp.abs(apq) < tiny
            tau = (aqq - app) / (2.0 * jnp.where(guard, one, apq))
            t = jnp.sign(tau) / (jnp.abs(tau) + jnp.sqrt(1.0 + tau * tau))
            t = jnp.where(guard, zero, t)
            c = 1.0 / jnp.sqrt(1.0 + t * t)
            s = t * c
            sa(p, p, app - t * apq)
            sa(q, q, aqq + t * apq)
            sa(p, q, zero)
            for k in range(4):
                if k == p or k == q:
                    continue
                akp, akq = ga(k, p), ga(k, q)
                sa(k, p, c * akp - s * akq)
                sa(k, q, s * akp + c * akq)
            for k in range(4):
                vkp, vkq = v[k][p], v[k][q]
                v[k][p] = c * vkp - s * vkq
                v[k][q] = s * vkp + c * vkq
    best_lam = ga(0, 0)
    qv = [v[k][0] for k in range(4)]
    for i in range(1, 4):
        lam_i = ga(i, i)
        cond = lam_i > best_lam
        qv = [jnp.where(cond, v[k][i], qv[k]) for k in range(4)]
        best_lam = jnp.where(cond, lam_i, best_lam)
    inv_n = 1.0 / jnp.sqrt(qv[0] ** 2 + qv[1] ** 2 + qv[2] ** 2 + qv[3] ** 2)
    qw, qx, qy, qz = (c_ * inv_n for c_ in qv)
    R = jnp.stack([
        jnp.stack([1 - 2 * (qy * qy + qz * qz), 2 * (qx * qy - qw * qz),
                   2 * (qx * qz + qw * qy)], axis=-1),
        jnp.stack([2 * (qx * qy + qw * qz), 1 - 2 * (qx * qx + qz * qz),
                   2 * (qy * qz - qw * qx)], axis=-1),
        jnp.stack([2 * (qx * qz - qw * qy), 2 * (qy * qz + qw * qx),
                   1 - 2 * (qx * qx + qy * qy)], axis=-1),
    ], axis=-2)
    return R


def kernel(src_embedding, tgt_embedding, src, tgt, temperature, is_corr):
    B, D, N = src_embedding.shape
    bm = _BM
    nb = N // bm
    src_bf = src_embedding.astype(jnp.bfloat16)
    tgt_bf = tgt_embedding.astype(jnp.bfloat16)
    ones = jnp.ones((B, 1, N), jnp.float32)
    zeros = jnp.zeros((B, 4, N), jnp.float32)
    src_ext = jnp.concatenate([src, ones, zeros], axis=1)
    tgt_ext = jnp.concatenate([tgt, ones, zeros], axis=1)
    tgt_hi = tgt_ext.astype(jnp.bfloat16)
    tgt_lo = (tgt_ext - tgt_hi.astype(jnp.float32)).astype(jnp.bfloat16)
    tgt16 = jnp.concatenate([tgt_hi, tgt_lo], axis=1)

    corres2, weight2, yg = pl.pallas_call(
        _scores_body,
        grid=(B, nb),
        in_specs=[
            pl.BlockSpec((B, 1), lambda b, n: (0, 0),
                         memory_space=pltpu.SMEM),
            pl.BlockSpec((1, D, bm), lambda b, n: (b, 0, n)),
            pl.BlockSpec((1, D, N), lambda b, n: (b, 0, 0)),
            pl.BlockSpec((1, 16, N), lambda b, n: (b, 0, 0)),
        ],
        out_specs=[
            pl.BlockSpec((1, bm, 1), lambda b, n: (b, n, 0)),
            pl.BlockSpec((1, bm, 1), lambda b, n: (b, n, 0)),
            pl.BlockSpec((1, 8, bm), lambda b, n: (b, 0, n)),
        ],
        out_shape=[
            jax.ShapeDtypeStruct((B, N, 1), jnp.int32),
            jax.ShapeDtypeStruct((B, N, 1), jnp.float32),
            jax.ShapeDtypeStruct((B, 8, N), jnp.float32),
        ],
        compiler_params=pltpu.CompilerParams(
            dimension_semantics=("parallel", "arbitrary")),
    )(temperature.astype(jnp.float32).reshape(B, 1), src_bf, tgt_bf,
      tgt16)

    cov8, mu8 = pl.pallas_call(
        _moments_body,
        grid=(B,),
        in_specs=[
            pl.BlockSpec((1, 8, N), lambda b: (b, 0, 0)),
            pl.BlockSpec((1, 8, N), lambda b: (b, 0, 0)),
            pl.BlockSpec((1, 1, N), lambda b: (b, 0, 0)),
        ],
        out_specs=[
            pl.BlockSpec((1, 8, 8), lambda b: (b, 0, 0)),
            pl.BlockSpec((1, 8, 8), lambda b: (b, 0, 0)),
        ],
        out_shape=[
            jax.ShapeDtypeStruct((B, 8, 8), jnp.float32),
            jax.ShapeDtypeStruct((B, 8, 8), jnp.float32),
        ],
        compiler_params=pltpu.CompilerParams(
            dimension_semantics=("arbitrary",)),
    )(yg, src_ext, weight2.reshape(B, 1, N))

    cov = cov8[:, :3, :3]
    mu_y = mu8[:, :3, 0]
    mu_x = mu8[:, :3, 1]
    R = _best_rotation(cov)
    rmux = jnp.sum(R * mu_x[:, None, :], axis=2)
    T = (mu_y[:, None, :] - rmux[:, :, None]).astype(jnp.float32)
    return (R.astype(jnp.float32), T, corres2, weight2)
